# baseline (speedup 1.0000x reference)
import functools

import jax
import jax.numpy as jnp
from jax.experimental import pallas as pl
from jax.experimental.pallas import tpu as pltpu

_L = 128


def _nll_body(conf_ref, tgt_ref, nll_ref, *, C):
    x = conf_ref[...][0]
    xt = x.T
    E = jnp.exp(xt)
    sums = jnp.sum(E, axis=0, keepdims=True)
    t = tgt_ref[...][0]
    riota = jax.lax.broadcasted_iota(jnp.int32, xt.shape, 0)
    g = jnp.sum(jnp.where(riota == t, xt, 0.0), axis=0, keepdims=True)
    nll_ref[...] = (jnp.log(sums) - g)[None]


def _sl1_body(loc_ref, loct_ref, tgt_ref, out_ref):
    d = (loc_ref[...][0] - loct_ref[...][0]).T
    a = jnp.abs(d)
    sl1 = jnp.where(a < 1.0, 0.5 * d * d, a - 0.5)
    pos = tgt_ref[...][0] > 0
    s = jnp.sum(jnp.where(pos, sl1, 0.0))
    lane0 = jax.lax.broadcasted_iota(jnp.int32, (1, _L), 1) == 0
    out_ref[...] = jnp.where(lane0, s, 0.0)[None]


def _select_body(nll_ref, tgt_ref, sl1_ref, conf_out, loc_out, tot_out,
                 lossc_ref, *, A):
    ct = tgt_ref[...]
    nll = nll_ref[...]
    pos = ct > 0
    lossc_ref[...] = jnp.where(pos, 0.0, jnp.maximum(nll, 0.0))

    lc0 = lossc_ref[...]
    p = jnp.sum(pos.astype(jnp.int32), axis=(1, 2), keepdims=True)
    k = jnp.minimum(3 * p, A - 1)
    G0 = jnp.sum((lc0 > 0.0).astype(jnp.int32), axis=(1, 2), keepdims=True)
    S0 = jnp.sum(lc0, axis=(1, 2), keepdims=True)

    def easy_fn(_):
        return jnp.zeros_like(S0), G0, S0

    def hard_fn(_):
        def body(_, carry):
            lo, hi = carry
            mid = lo + ((hi - lo) >> 1)
            thr = jax.lax.bitcast_convert_type(mid, jnp.float32)
            cnt = jnp.sum((lossc_ref[...] >= thr).astype(jnp.int32),
                          axis=(1, 2), keepdims=True)
            big = cnt >= k
            return jnp.where(big, mid, lo), jnp.where(big, hi, mid)

        lo0 = jnp.zeros_like(p)
        hi0 = jnp.full_like(p, 0x7F800001)
        lo, _ = jax.lax.fori_loop(0, 31, body, (lo0, hi0))
        T = jax.lax.bitcast_convert_type(lo, jnp.float32)
        lc = lossc_ref[...]
        gt = lc > T
        G = jnp.sum(gt.astype(jnp.int32), axis=(1, 2), keepdims=True)
        S1 = jnp.sum(jnp.where(gt, lc, 0.0), axis=(1, 2), keepdims=True)
        return T, G, S1

    easy = jnp.all(k >= G0)
    T, G, S1 = jax.lax.cond(easy, easy_fn, hard_fn, None)

    m = (k - G).astype(jnp.float32)
    negsum = S1 + jnp.where(m > 0, m * T, 0.0)
    posnll = jnp.sum(jnp.where(pos, nll, 0.0), axis=(1, 2), keepdims=True)

    N = jnp.sum(p).astype(jnp.float32)
    lc_v = jnp.sum(posnll + negsum) / N
    ll_v = jnp.sum(sl1_ref[...]) / N
    conf_out[0, 0] = lc_v
    loc_out[0, 0] = ll_v
    tot_out[0, 0] = lc_v + ll_v


def kernel(loc, conf, loc_target, conf_target):
    b, A, C = conf.shape
    ct = conf_target.astype(jnp.int32)
    ct3 = ct.reshape(b, 1, A)

    nll = pl.pallas_call(
        functools.partial(_nll_body, C=C),
        grid=(b,),
        in_specs=[
            pl.BlockSpec((1, A, C), lambda i: (i, 0, 0)),
            pl.BlockSpec((1, 1, A), lambda i: (i, 0, 0)),
        ],
        out_specs=pl.BlockSpec((1, 1, A), lambda i: (i, 0, 0)),
        out_shape=jax.ShapeDtypeStruct((b, 1, A), jnp.float32),
    )(conf, ct3)

    sl1p = pl.pallas_call(
        _sl1_body,
        grid=(b,),
        in_specs=[
            pl.BlockSpec((1, A, 4), lambda i: (i, 0, 0)),
            pl.BlockSpec((1, A, 4), lambda i: (i, 0, 0)),
            pl.BlockSpec((1, 1, A), lambda i: (i, 0, 0)),
        ],
        out_specs=pl.BlockSpec((1, 1, _L), lambda i: (i, 0, 0)),
        out_shape=jax.ShapeDtypeStruct((b, 1, _L), jnp.float32),
    )(loc, loc_target, ct3)

    outs = pl.pallas_call(
        functools.partial(_select_body, A=A),
        in_specs=[
            pl.BlockSpec((b, 1, A), lambda: (0, 0, 0)),
            pl.BlockSpec((b, 1, A), lambda: (0, 0, 0)),
            pl.BlockSpec((b, _L), lambda: (0, 0)),
        ],
        out_specs=[
            pl.BlockSpec(memory_space=pltpu.SMEM),
            pl.BlockSpec(memory_space=pltpu.SMEM),
            pl.BlockSpec(memory_space=pltpu.SMEM),
        ],
        out_shape=[jax.ShapeDtypeStruct((1, 1), jnp.float32)] * 3,
        scratch_shapes=[pltpu.VMEM((b, 1, A), jnp.float32)],
    )(nll, ct3, sl1p.reshape(b, _L))

    return tuple(o.reshape(()) for o in outs)


# speedup vs baseline: 2.0563x; 2.0563x over previous
import functools

import numpy as np
import jax
import jax.numpy as jnp
from jax.experimental import pallas as pl
from jax.experimental.pallas import tpu as pltpu

_L = 128


def _nll_body(conf_ref, tgt_ref, nll_ref, *, C):
    x = conf_ref[...][0]
    xt = x.T
    E = jnp.exp(xt)
    sums = jnp.sum(E, axis=0, keepdims=True)
    t = tgt_ref[...][0]
    riota = jax.lax.broadcasted_iota(jnp.int32, xt.shape, 0)
    g = jnp.sum(jnp.where(riota == t, xt, 0.0), axis=0, keepdims=True)
    nll_ref[...] = (jnp.log(sums) - g)[None]


def _sl1_body(r4_ref, d_ref, tgt_ref, out_ref):
    d = d_ref[...][0]
    a = jnp.abs(d)
    sl1 = jnp.where(a < 1.0, 0.5 * d * d, a - 0.5)
    posf = jnp.dot((tgt_ref[...][0] > 0).astype(jnp.float32), r4_ref[...],
                   preferred_element_type=jnp.float32)
    s = jnp.sum(sl1 * posf)
    lane0 = jax.lax.broadcasted_iota(jnp.int32, (1, _L), 1) == 0
    out_ref[...] = jnp.where(lane0, s, 0.0)[None]


def _select_body(nll_ref, tgt_ref, sl1_ref, conf_out, loc_out, tot_out,
                 lossc_ref, *, A):
    ct = tgt_ref[...]
    nll = nll_ref[...]
    pos = ct > 0
    lossc_ref[...] = jnp.where(pos, 0.0, jnp.maximum(nll, 0.0))

    lc0 = lossc_ref[...]
    p = jnp.sum(pos.astype(jnp.int32), axis=(1, 2), keepdims=True)
    k = jnp.minimum(3 * p, A - 1)
    G0 = jnp.sum((lc0 > 0.0).astype(jnp.int32), axis=(1, 2), keepdims=True)
    S0 = jnp.sum(lc0, axis=(1, 2), keepdims=True)

    def easy_fn(_):
        return jnp.zeros_like(S0), G0, S0

    def hard_fn(_):
        def body(_, carry):
            lo, hi = carry
            mid = lo + ((hi - lo) >> 1)
            thr = jax.lax.bitcast_convert_type(mid, jnp.float32)
            cnt = jnp.sum((lossc_ref[...] >= thr).astype(jnp.int32),
                          axis=(1, 2), keepdims=True)
            big = cnt >= k
            return jnp.where(big, mid, lo), jnp.where(big, hi, mid)

        lo0 = jnp.zeros_like(p)
        hi0 = jnp.full_like(p, 0x7F800001)
        lo, _ = jax.lax.fori_loop(0, 31, body, (lo0, hi0))
        T = jax.lax.bitcast_convert_type(lo, jnp.float32)
        lc = lossc_ref[...]
        gt = lc > T
        G = jnp.sum(gt.astype(jnp.int32), axis=(1, 2), keepdims=True)
        S1 = jnp.sum(jnp.where(gt, lc, 0.0), axis=(1, 2), keepdims=True)
        return T, G, S1

    easy = jnp.all(k >= G0)
    T, G, S1 = jax.lax.cond(easy, easy_fn, hard_fn, None)

    m = (k - G).astype(jnp.float32)
    negsum = S1 + jnp.where(m > 0, m * T, 0.0)
    posnll = jnp.sum(jnp.where(pos, nll, 0.0), axis=(1, 2), keepdims=True)

    N = jnp.sum(p).astype(jnp.float32)
    lc_v = jnp.sum(posnll + negsum) / N
    ll_v = jnp.sum(sl1_ref[...]) / N
    conf_out[0, 0] = lc_v
    loc_out[0, 0] = ll_v
    tot_out[0, 0] = lc_v + ll_v


def kernel(loc, conf, loc_target, conf_target):
    b, A, C = conf.shape
    ct = conf_target.astype(jnp.int32)
    ct3 = ct.reshape(b, 1, A)

    nll = pl.pallas_call(
        functools.partial(_nll_body, C=C),
        grid=(b,),
        in_specs=[
            pl.BlockSpec((1, A, C), lambda i: (i, 0, 0)),
            pl.BlockSpec((1, 1, A), lambda i: (i, 0, 0)),
        ],
        out_specs=pl.BlockSpec((1, 1, A), lambda i: (i, 0, 0)),
        out_shape=jax.ShapeDtypeStruct((b, 1, A), jnp.float32),
    )(conf, ct3)

    LB = (A * 4) // _L
    AW = _L // 4
    d2 = (loc - loc_target).reshape(b, LB, _L)
    ct_loc = ct.reshape(b, LB, AW)
    R4 = jnp.asarray(np.repeat(np.eye(AW, dtype=np.float32), 4, axis=1))

    sl1p = pl.pallas_call(
        _sl1_body,
        grid=(b,),
        in_specs=[
            pl.BlockSpec((AW, _L), lambda i: (0, 0)),
            pl.BlockSpec((1, LB, _L), lambda i: (i, 0, 0)),
            pl.BlockSpec((1, LB, AW), lambda i: (i, 0, 0)),
        ],
        out_specs=pl.BlockSpec((1, 1, _L), lambda i: (i, 0, 0)),
        out_shape=jax.ShapeDtypeStruct((b, 1, _L), jnp.float32),
    )(R4, d2, ct_loc)

    outs = pl.pallas_call(
        functools.partial(_select_body, A=A),
        in_specs=[
            pl.BlockSpec((b, 1, A), lambda: (0, 0, 0)),
            pl.BlockSpec((b, 1, A), lambda: (0, 0, 0)),
            pl.BlockSpec((b, _L), lambda: (0, 0)),
        ],
        out_specs=[
            pl.BlockSpec(memory_space=pltpu.SMEM),
            pl.BlockSpec(memory_space=pltpu.SMEM),
            pl.BlockSpec(memory_space=pltpu.SMEM),
        ],
        out_shape=[jax.ShapeDtypeStruct((1, 1), jnp.float32)] * 3,
        scratch_shapes=[pltpu.VMEM((b, 1, A), jnp.float32)],
    )(nll, ct3, sl1p.reshape(b, _L))

    return tuple(o.reshape(()) for o in outs)
